# baseline (speedup 1.0000x reference)
import math

import jax
import jax.numpy as jnp
from jax import lax
from jax.experimental import pallas as pl
from jax.experimental.pallas import tpu as pltpu
from jax.experimental.pallas import tpu_sc as plsc

VOCAB = 1000000
EMB = 64
B = 4096
L = 200
SCALE = math.sqrt(EMB)

NC = 2
NS = 16
NW = NC * NS
LANES = 16

TOTAL = B * L
PER_W = TOTAL // NW
CHUNK = 128
NCHUNK = PER_W // CHUNK


def _body(table_hbm, idx_hbm, out_hbm, idx_v, rows_v, sem):
  wid = lax.axis_index("s") * NC + lax.axis_index("c")
  pltpu.sync_copy(idx_hbm.at[wid], idx_v)
  base = wid * PER_W

  def chunk_body(c, carry):
    pltpu.async_copy(table_hbm.at[idx_v.at[c]], rows_v, sem).wait()

    def scale_body(i, c2):
      for j in range(EMB // LANES):
        sl = pl.ds(j * LANES, LANES)
        rows_v[i, sl] = rows_v[i, sl] * SCALE
      return c2

    lax.fori_loop(0, CHUNK, scale_body, 0, unroll=4)

    pltpu.sync_copy(rows_v, out_hbm.at[pl.ds(base + c * CHUNK, CHUNK)])
    return carry

  lax.fori_loop(0, NCHUNK, chunk_body, 0)


@jax.jit
def _embed(tokens, embedding_weight):
  idx = jnp.reshape(tokens.astype(jnp.int32), (NW, NCHUNK, CHUNK))
  mesh = plsc.VectorSubcoreMesh(core_axis_name="c", subcore_axis_name="s")
  kfn = pl.kernel(
      _body,
      out_type=jax.ShapeDtypeStruct((TOTAL, EMB), jnp.float32),
      mesh=mesh,
      scratch_types=[
          pltpu.VMEM((NCHUNK, CHUNK), jnp.int32),
          pltpu.VMEM((CHUNK, EMB), jnp.float32),
          pltpu.SemaphoreType.DMA,
      ],
      compiler_params=pltpu.CompilerParams(use_tc_tiling_on_sc=False),
  )
  out = kfn(embedding_weight, idx)
  return jnp.reshape(out, (B, L, EMB))


def kernel(tokens, embedding_weight):
  return _embed(tokens, embedding_weight)


# speedup vs baseline: 1.0539x; 1.0539x over previous
import math

import jax
import jax.numpy as jnp
from jax import lax
from jax.experimental import pallas as pl
from jax.experimental.pallas import tpu as pltpu
from jax.experimental.pallas import tpu_sc as plsc

VOCAB = 1000000
EMB = 64
B = 4096
L = 200
SCALE = math.sqrt(EMB)

NC = 2
NS = 16
NW = NC * NS
LANES = 16

TOTAL = B * L
PER_W = TOTAL // NW
CHUNK = 128
NCHUNK = PER_W // CHUNK


NBUF = 4
NGROUP = NCHUNK // NBUF


def _body(table_hbm, idx_hbm, out_hbm, idx_v, in_v, out_v, gsems, ssems):
  wid = lax.axis_index("s") * NC + lax.axis_index("c")
  pltpu.sync_copy(idx_hbm.at[wid], idx_v)
  base = wid * PER_W

  for b in range(NBUF):
    pltpu.async_copy(table_hbm.at[idx_v.at[b]], in_v.at[b], gsems[b])

  def group_body(g, carry):
    for b in range(NBUF):
      c = g * NBUF + b
      pltpu.make_async_copy(table_hbm.at[idx_v.at[b]], in_v.at[b], gsems[b]).wait()
      @pl.when(g > 0)
      def _():
        pltpu.make_async_copy(
            out_v.at[b], out_hbm.at[pl.ds(base, CHUNK)], ssems[b]).wait()

      def scale_body(i, c2):
        for j in range(EMB // LANES):
          sl = pl.ds(j * LANES, LANES)
          out_v[b, i, sl] = in_v[b, i, sl] * SCALE
        return c2

      lax.fori_loop(0, CHUNK, scale_body, 0, unroll=4)

      @pl.when(g < NGROUP - 1)
      def _():
        pltpu.async_copy(
            table_hbm.at[idx_v.at[c + NBUF]], in_v.at[b], gsems[b])

      pltpu.async_copy(
          out_v.at[b], out_hbm.at[pl.ds(base + c * CHUNK, CHUNK)], ssems[b])
    return carry

  lax.fori_loop(0, NGROUP, group_body, 0)

  for b in range(NBUF):
    pltpu.make_async_copy(
        out_v.at[b], out_hbm.at[pl.ds(base, CHUNK)], ssems[b]).wait()


@jax.jit
def _embed(tokens, embedding_weight):
  idx = jnp.reshape(tokens.astype(jnp.int32), (NW, NCHUNK, CHUNK))
  mesh = plsc.VectorSubcoreMesh(core_axis_name="c", subcore_axis_name="s")
  kfn = pl.kernel(
      _body,
      out_type=jax.ShapeDtypeStruct((TOTAL, EMB), jnp.float32),
      mesh=mesh,
      scratch_types=[
          pltpu.VMEM((NCHUNK, CHUNK), jnp.int32),
          pltpu.VMEM((NBUF, CHUNK, EMB), jnp.float32),
          pltpu.VMEM((NBUF, CHUNK, EMB), jnp.float32),
          [pltpu.SemaphoreType.DMA] * NBUF,
          [pltpu.SemaphoreType.DMA] * NBUF,
      ],
      compiler_params=pltpu.CompilerParams(use_tc_tiling_on_sc=False),
  )
  out = kfn(embedding_weight, idx)
  return jnp.reshape(out, (B, L, EMB))


def kernel(tokens, embedding_weight):
  return _embed(tokens, embedding_weight)


# speedup vs baseline: 1.0570x; 1.0030x over previous
import math

import jax
import jax.numpy as jnp
from jax import lax
from jax.experimental import pallas as pl
from jax.experimental.pallas import tpu as pltpu
from jax.experimental.pallas import tpu_sc as plsc

VOCAB = 1000000
EMB = 64
B = 4096
L = 200
SCALE = math.sqrt(EMB)

NC = 2
NS = 16
NW = NC * NS
LANES = 16

ROWS_W = B // NW
NBUF = 4
NGROUP = ROWS_W // NBUF


def _body(table_hbm, idx_hbm, out_hbm, idx_v, in_v, out_v, gsems, ssems):
  wid = lax.axis_index("s") * NC + lax.axis_index("c")
  row0 = wid * ROWS_W
  pltpu.sync_copy(idx_hbm.at[pl.ds(row0, ROWS_W)], idx_v)

  def gather_start(b, r):
    pltpu.async_copy(table_hbm.at[idx_v.at[r]], in_v.at[b], gsems[b])

  def gather_wait(b):
    pltpu.make_async_copy(table_hbm.at[idx_v.at[0]], in_v.at[b], gsems[b]).wait()

  def store_start(b, r):
    pltpu.async_copy(out_v.at[b], out_hbm.at[row0 + r], ssems[b])

  def store_wait(b):
    pltpu.make_async_copy(out_v.at[b], out_hbm.at[row0], ssems[b]).wait()

  for b in range(NBUF):
    gather_start(b, b)

  def group_body(g, carry):
    for b in range(NBUF):
      r = g * NBUF + b
      gather_wait(b)
      @pl.when(g > 0)
      def _():
        store_wait(b)

      def scale_body(i, c2):
        for j in range(EMB // LANES):
          sl = pl.ds(j * LANES, LANES)
          out_v[b, i, sl] = in_v[b, i, sl] * SCALE
        return c2

      lax.fori_loop(0, L, scale_body, 0, unroll=4)

      @pl.when(r + NBUF < ROWS_W)
      def _():
        gather_start(b, r + NBUF)

      store_start(b, r)
    return carry

  lax.fori_loop(0, NGROUP, group_body, 0)

  for b in range(NBUF):
    store_wait(b)


@jax.jit
def _embed(tokens, embedding_weight):
  mesh = plsc.VectorSubcoreMesh(core_axis_name="c", subcore_axis_name="s")
  kfn = pl.kernel(
      _body,
      out_type=jax.ShapeDtypeStruct((B, L, EMB), jnp.float32),
      mesh=mesh,
      scratch_types=[
          pltpu.VMEM((ROWS_W, L), jnp.int32),
          pltpu.VMEM((NBUF, L, EMB), jnp.float32),
          pltpu.VMEM((NBUF, L, EMB), jnp.float32),
          [pltpu.SemaphoreType.DMA] * NBUF,
          [pltpu.SemaphoreType.DMA] * NBUF,
      ],
      compiler_params=pltpu.CompilerParams(use_tc_tiling_on_sc=False),
  )
  return kfn(embedding_weight, tokens.astype(jnp.int32))


def kernel(tokens, embedding_weight):
  return _embed(tokens, embedding_weight)
